# baseline (speedup 1.0000x reference)
import jax
import jax.numpy as jnp
from jax.experimental import pallas as pl
from jax.experimental.pallas import tpu as pltpu


def _copy_kernel(experts_ref, scales_ref, experts_out_ref, scales_out_ref,
                 sem_e, sem_s):
    copy_e = pltpu.make_async_copy(experts_ref, experts_out_ref, sem_e)
    copy_s = pltpu.make_async_copy(scales_ref, scales_out_ref, sem_s)
    copy_e.start()
    copy_s.start()
    copy_e.wait()
    copy_s.wait()


def kernel(router_logits, routing_tensor, routing_scales):
    del router_logits
    n_tokens, top_k = routing_tensor.shape
    n_tiles = n_tokens // 128
    experts3 = routing_tensor.reshape(n_tiles, 128, top_k).transpose(0, 2, 1)
    scales3 = routing_scales.reshape(n_tiles, 128, top_k).transpose(0, 2, 1)
    experts_out, scales_out = pl.pallas_call(
        _copy_kernel,
        in_specs=[
            pl.BlockSpec(memory_space=pl.ANY),
            pl.BlockSpec(memory_space=pl.ANY),
        ],
        out_specs=(
            pl.BlockSpec(memory_space=pl.ANY),
            pl.BlockSpec(memory_space=pl.ANY),
        ),
        out_shape=(
            jax.ShapeDtypeStruct((n_tiles, top_k, 128), routing_tensor.dtype),
            jax.ShapeDtypeStruct((n_tiles, top_k, 128), routing_scales.dtype),
        ),
        scratch_shapes=[pltpu.SemaphoreType.DMA, pltpu.SemaphoreType.DMA],
    )(experts3, scales3)
    return (
        experts_out.transpose(0, 2, 1).reshape(n_tokens, top_k),
        scales_out.transpose(0, 2, 1).reshape(n_tokens, top_k),
    )


# speedup vs baseline: 1.8121x; 1.8121x over previous
import jax
import jax.numpy as jnp
from jax.experimental import pallas as pl
from jax.experimental.pallas import tpu as pltpu


def _copy_kernel(experts_ref, scales_ref, experts_out_ref, scales_out_ref):
    experts_out_ref[...] = experts_ref[...]
    scales_out_ref[...] = scales_ref[...]


def kernel(router_logits, routing_tensor, routing_scales):
    del router_logits
    n_tokens, top_k = routing_tensor.shape
    n_tiles = n_tokens // 128
    experts3 = routing_tensor.reshape(n_tiles, 128, top_k).transpose(0, 2, 1)
    scales3 = routing_scales.reshape(n_tiles, 128, top_k).transpose(0, 2, 1)
    experts_out, scales_out = pl.pallas_call(
        _copy_kernel,
        compiler_params=pltpu.CompilerParams(skip_device_barrier=True),
        out_shape=(
            jax.ShapeDtypeStruct((n_tiles, top_k, 128), routing_tensor.dtype),
            jax.ShapeDtypeStruct((n_tiles, top_k, 128), routing_scales.dtype),
        ),
    )(experts3, scales3)
    return (
        experts_out.transpose(0, 2, 1).reshape(n_tokens, top_k),
        scales_out.transpose(0, 2, 1).reshape(n_tokens, top_k),
    )
